# baseline (speedup 1.0000x reference)
import functools

import jax
import jax.numpy as jnp
from jax import lax
from jax.experimental import pallas as pl
from jax.experimental.pallas import tpu as pltpu
from jax.experimental.pallas import tpu_sc as plsc

N = 131072
K = 64
M = 16
R = 16384
NB = N // R
ALPHA = 0.7
BETA = 1.5

NC = 2
NS = 16
NW = NC * NS
ROWS_W = N // NW
CHUNK = 512
IDXROWS = ROWS_W // 128


def _sc_p1(true_hbm, emb_hbm, sum_out, cnt_out,
           ebuf, tbuf, ones_b, zbuf, zcnt, shr_sum, shr_cnt):
    c = lax.axis_index("c")
    s = lax.axis_index("s")
    wid = c * NS + s
    base = wid * ROWS_W

    for i in range(M):
        for j in range(K // 16):
            zbuf[i, pl.ds(j * 16, 16)] = jnp.zeros((16,), jnp.float32)
        zcnt[i] = jnp.zeros((16,), jnp.float32)
    for i in range(128):
        ones_b[i] = jnp.ones((16,), jnp.float32)

    @pl.when(s == 0)
    def _():
        pltpu.sync_copy(zbuf, shr_sum)
        pltpu.sync_copy(zcnt, shr_cnt)

    plsc.subcore_barrier()

    pltpu.sync_copy(true_hbm.at[pl.ds(wid * IDXROWS, IDXROWS)], tbuf)

    for ch in range(ROWS_W // CHUNK):
        pltpu.sync_copy(emb_hbm.at[pl.ds(base + ch * CHUNK, CHUNK)], ebuf)
        for j in range(CHUNK // 128):
            idx = tbuf.at[ch * (CHUNK // 128) + j]
            pltpu.sync_copy(ebuf.at[pl.ds(j * 128, 128)],
                            shr_sum.at[idx], add=True)
            pltpu.sync_copy(ones_b, shr_cnt.at[idx], add=True)

    plsc.subcore_barrier()

    @pl.when(s == 0)
    def _():
        pltpu.sync_copy(shr_sum, sum_out.at[c])
        pltpu.sync_copy(shr_cnt, cnt_out.at[c])


def _sc_pass1(true2, emb2):
    mesh = plsc.VectorSubcoreMesh(core_axis_name="c", subcore_axis_name="s")
    kfn = functools.partial(
        pl.kernel,
        mesh=mesh,
        out_type=[
            jax.ShapeDtypeStruct((NC, M, K), jnp.float32),
            jax.ShapeDtypeStruct((NC, M, 16), jnp.float32),
        ],
        scratch_types=[
            pltpu.VMEM((CHUNK, K), jnp.float32),
            pltpu.VMEM((IDXROWS, 128), jnp.int32),
            pltpu.VMEM((128, M), jnp.float32),
            pltpu.VMEM((M, K), jnp.float32),
            pltpu.VMEM((M, 16), jnp.float32),
            pltpu.VMEM_SHARED((M, K), jnp.float32),
            pltpu.VMEM_SHARED((M, 16), jnp.float32),
        ],
    )(_sc_p1)
    return kfn(true2, emb2)


def _p2_body(true_l_ref, emb_ref, pts_ref, sump_ref, cntp_ref,
             out_ref, mean_s, wrow_s, acc_s):
    step = pl.program_id(0)

    @pl.when(step == 0)
    def _():
        seg = sump_ref[0] + sump_ref[1]
        cnt16 = cntp_ref[0] + cntp_ref[1]
        mean_s[...] = seg / cnt16[:, :1]
        ii = jax.lax.broadcasted_iota(jnp.int32, (M, M), 0)
        jj = jax.lax.broadcasted_iota(jnp.int32, (M, M), 1)
        eye = (ii == jj).astype(jnp.float32)
        cnt_t = jax.lax.dot_general(
            cnt16, eye, (((0,), (0,)), ((), ())),
            preferred_element_type=jnp.float32)
        lane_ids = jax.lax.broadcasted_iota(jnp.int32, (1, M), 1)
        labmask = (lane_ids >= 1).astype(jnp.float32)
        wrow_s[...] = labmask / cnt_t[:1, :]
        acc_s[...] = jnp.zeros_like(acc_s)
        out_ref[...] = jnp.zeros_like(out_ref)

    lab = true_l_ref[...]
    oh_t = (lab == jax.lax.broadcasted_iota(jnp.int32, (M, 1), 0)
            ).astype(jnp.float32)

    emb = emb_ref[0]
    dt = jax.lax.dot_general(
        mean_s[...], emb, (((1,), (1,)), ((), ())),
        preferred_element_type=jnp.float32)
    dot_own = jnp.sum(oh_t * dt, axis=0, keepdims=True)
    sq = emb * emb
    e2 = jax.lax.dot_general(
        jnp.ones((1, K), jnp.float32), sq, (((1,), (1,)), ((), ())),
        preferred_element_type=jnp.float32)
    m2 = jnp.sum(mean_s[...] * mean_s[...], axis=1, keepdims=True)
    m2_own = jnp.sum(oh_t * m2, axis=0, keepdims=True)
    d2 = jnp.maximum(e2 - 2.0 * dot_own + m2_own, 0.0)
    d = jnp.sqrt(d2)

    pts = pts_ref[0]
    psq = jax.lax.dot_general(
        jnp.ones((1, 3), jnp.float32),
        pts * pts, (((1,), (1,)), ((), ())),
        preferred_element_type=jnp.float32)
    g = jax.nn.sigmoid(jnp.sqrt(psq))

    w = jax.lax.dot_general(
        wrow_s[...], oh_t, (((1,), (0,)), ((), ())),
        preferred_element_type=jnp.float32)
    hinge = jnp.maximum(d - ALPHA, 0.0)
    acc_s[...] += g * hinge * hinge * w

    @pl.when(step == NB - 1)
    def _():
        intra = jnp.sum(acc_s[...])

        m = mean_s[...]
        gram = jax.lax.dot_general(
            m, m, (((1,), (1,)), ((), ())),
            preferred_element_type=jnp.float32)
        ii = jax.lax.broadcasted_iota(jnp.int32, (M, M), 0)
        jj = jax.lax.broadcasted_iota(jnp.int32, (M, M), 1)
        diag = (ii == jj).astype(jnp.float32)
        nrm_col = jnp.sum(gram * diag, axis=1, keepdims=True)
        nrm_row = jnp.sum(gram * diag, axis=0, keepdims=True)
        d2p = jnp.maximum(nrm_col + nrm_row - 2.0 * gram, 0.0)
        dp = jnp.sqrt(d2p)
        hp = jnp.maximum(BETA - dp, 0.0)
        offdiag = ((ii != jj) & (ii >= 1) & (jj >= 1)).astype(jnp.float32)
        inter = jnp.sum(hp * hp * offdiag)

        val = intra / M + inter / (M * (M - 1))
        out_ref[...] = val.reshape(1, 1)


def kernel(points, true, embedding):
    true2 = true.reshape(N // 128, 128)
    emb2 = embedding.reshape(N, K)

    sum_parts, cnt_parts = _sc_pass1(true2, emb2)

    out = pl.pallas_call(
        _p2_body,
        grid=(NB,),
        in_specs=[
            pl.BlockSpec((1, R), lambda i: (0, i)),
            pl.BlockSpec((1, R, K), lambda i: (0, i, 0)),
            pl.BlockSpec((1, R, 3), lambda i: (0, i, 0)),
            pl.BlockSpec((NC, M, K), lambda i: (0, 0, 0)),
            pl.BlockSpec((NC, M, 16), lambda i: (0, 0, 0)),
        ],
        out_specs=pl.BlockSpec((1, 1), lambda i: (0, 0)),
        out_shape=jax.ShapeDtypeStruct((1, 1), jnp.float32),
        scratch_shapes=[
            pltpu.VMEM((M, K), jnp.float32),
            pltpu.VMEM((1, M), jnp.float32),
            pltpu.VMEM((1, R), jnp.float32),
        ],
    )(true, embedding, points, sum_parts, cnt_parts)

    return out.reshape(1)
